# Initial kernel scaffold; baseline (speedup 1.0000x reference)
import jax
import jax.numpy as jnp
from jax.experimental import pallas as pl


def kernel(X, edge_index, edge_weight, W_xz, W_qz, W_xr, W_qr, W_xh, W_qh, w_x_z, w_q_z, w_x_r, w_q_r, w_x_h, w_q_h, b_z, b_r, b_h):
    raise NotImplementedError("write your pallas kernel here")


# speedup vs baseline: 28.3541x; 28.3541x over previous
import jax
import jax.numpy as jnp
from jax.experimental import pallas as pl
from jax.experimental.pallas import tpu as pltpu

N = 24
E = 384
C = 512


def _gru_kernel(ei_ref, ew_ref, x_ref, wz_ref, wh_ref, uz_ref, uh_ref,
                bz_ref, bh_ref, out_ref):
    f32 = jnp.float32
    rowT = ei_ref[0:1, :]
    colT = ei_ref[1:2, :]
    ew = ew_ref[:]

    node_iota = jax.lax.broadcasted_iota(jnp.int32, (N, E), 0)
    ohr = (node_iota == rowT).astype(f32)
    ohc = (node_iota == colT).astype(f32)

    deg = jnp.dot(ohr, ew.T, preferred_element_type=f32)
    dinv = jnp.where(deg > 0.0, jax.lax.rsqrt(deg), 0.0)

    dinv_row = jnp.dot(dinv.T, ohr, preferred_element_type=f32)
    dinv_col = jnp.dot(dinv.T, ohc, preferred_element_type=f32)
    wn = -(dinv_row * ew * dinv_col)

    a = jax.lax.dot_general(ohr * wn, ohc, (((1,), (1,)), ((), ())),
                            preferred_element_type=f32)

    x = x_ref[:]
    tx1 = jnp.dot(a, x, preferred_element_type=f32)

    cz = jax.nn.relu(
        jnp.dot(x, wz_ref[0], preferred_element_type=f32)
        + jnp.dot(tx1, wz_ref[1], preferred_element_type=f32))
    ch = jax.nn.relu(
        jnp.dot(x, wh_ref[0], preferred_element_type=f32)
        + jnp.dot(tx1, wh_ref[1], preferred_element_type=f32))

    z = jax.nn.sigmoid(
        jax.lax.dot_general(cz, uz_ref[:], (((1,), (1,)), ((), ())),
                            preferred_element_type=f32) + bz_ref[:])
    ht = jnp.tanh(
        jax.lax.dot_general(ch, uh_ref[:], (((1,), (1,)), ((), ())),
                            preferred_element_type=f32) + bh_ref[:])
    out_ref[:] = z * ht


def kernel(X, edge_index, edge_weight, W_xz, W_qz, W_xr, W_qr, W_xh, W_qh,
           w_x_z, w_q_z, w_x_r, w_q_r, w_x_h, w_q_h, b_z, b_r, b_h):
    ei = edge_index.astype(jnp.int32)
    ew = edge_weight.reshape(1, E).astype(jnp.float32)
    return pl.pallas_call(
        _gru_kernel,
        out_shape=jax.ShapeDtypeStruct((N, C), jnp.float32),
    )(ei, ew, X, W_xz, W_xh, w_x_z, w_x_h, b_z, b_h)
